# Initial kernel scaffold; baseline (speedup 1.0000x reference)
import jax
import jax.numpy as jnp
from jax.experimental import pallas as pl


def kernel(x, edge_index, W1, b1, W2, b2, W3, b3, Wr1, br1, Wr2, br2):
    raise NotImplementedError("write your pallas kernel here")


# speedup vs baseline: 84.4943x; 84.4943x over previous
import jax
import jax.numpy as jnp
from jax import lax
from jax.experimental import pallas as pl
from jax.experimental.pallas import tpu as pltpu
from jax.experimental.pallas import tpu_sc as plsc

NC = 2
NS = 16
NW = NC * NS
GRP = 128
GPC = 8
H = 8
RB = 2048


def _sc_degree(dst2, zeros_n, n_pad, cpw):
    rows_n = n_pad // NS

    def body(dst_hbm, zero_hbm, out_hbm, deg_sh, onesv, idxv, sem):
        c = lax.axis_index("c")
        s = lax.axis_index("s")
        w = c * NS + s
        sl = pl.ds(s * rows_n, rows_n)
        pltpu.sync_copy(zero_hbm.at[sl], deg_sh.at[sl])
        for i in range(GRP // 16):
            onesv[pl.ds(i * 16, 16)] = jnp.ones((16,), jnp.float32)
        plsc.subcore_barrier()

        @pl.loop(0, cpw)
        def _chunk(ci):
            row0 = (w * cpw + ci) * GPC
            pltpu.sync_copy(dst_hbm.at[pl.ds(row0, GPC)], idxv)
            ds_ = [pltpu.async_copy(onesv, deg_sh.at[idxv.at[g]], sem, add=True)
                   for g in range(GPC)]
            for d in ds_:
                d.wait()

        plsc.subcore_barrier()
        pltpu.sync_copy(deg_sh.at[sl], out_hbm.at[c, sl])

    f = pl.kernel(
        body,
        out_type=jax.ShapeDtypeStruct((NC, n_pad), jnp.float32),
        mesh=plsc.VectorSubcoreMesh(core_axis_name="c", subcore_axis_name="s"),
        scratch_types=[
            pltpu.VMEM_SHARED((n_pad,), jnp.float32),
            pltpu.VMEM((GRP,), jnp.float32),
            pltpu.VMEM((GPC, GRP), jnp.int32),
            pltpu.SemaphoreType.DMA,
        ],
    )
    return f(dst2, zeros_n)


def _sc_edge(u, src2, dst2, zeros_n8, n_pad, cpw):
    rows_n = n_pad // NS

    def body(u_hbm, src_hbm, dst_hbm, zero_hbm, out_hbm,
             u_sh, acc_sh, srcv, dstv, rowsv, gsem, ssem):
        c = lax.axis_index("c")
        s = lax.axis_index("s")
        w = c * NS + s
        sl = pl.ds(s * rows_n, rows_n)
        pltpu.sync_copy(u_hbm.at[sl], u_sh.at[sl])
        pltpu.sync_copy(zero_hbm.at[sl], acc_sh.at[sl])
        plsc.subcore_barrier()

        @pl.loop(0, cpw)
        def _chunk(ci):
            row0 = (w * cpw + ci) * GPC
            pltpu.sync_copy(src_hbm.at[pl.ds(row0, GPC)], srcv)
            pltpu.sync_copy(dst_hbm.at[pl.ds(row0, GPC)], dstv)
            gs = [pltpu.async_copy(u_sh.at[srcv.at[g]], rowsv.at[g], gsem)
                  for g in range(GPC)]
            for d in gs:
                d.wait()
            ss = [pltpu.async_copy(rowsv.at[g], acc_sh.at[dstv.at[g]], ssem, add=True)
                  for g in range(GPC)]
            for d in ss:
                d.wait()

        plsc.subcore_barrier()
        pltpu.sync_copy(acc_sh.at[sl], out_hbm.at[c, sl])

    f = pl.kernel(
        body,
        out_type=jax.ShapeDtypeStruct((NC, n_pad, H), jnp.float32),
        mesh=plsc.VectorSubcoreMesh(core_axis_name="c", subcore_axis_name="s"),
        scratch_types=[
            pltpu.VMEM_SHARED((n_pad, H), jnp.float32),
            pltpu.VMEM_SHARED((n_pad, H), jnp.float32),
            pltpu.VMEM((GPC, GRP), jnp.int32),
            pltpu.VMEM((GPC, GRP), jnp.int32),
            pltpu.VMEM((GPC, GRP, H), jnp.float32),
            pltpu.SemaphoreType.DMA,
            pltpu.SemaphoreType.DMA,
        ],
        compiler_params=pltpu.CompilerParams(use_tc_tiling_on_sc=False),
    )
    return f(u, src2, dst2, zeros_n8)


def _elu(t):
    return jnp.where(t > 0, t, jnp.exp(t) - 1.0)


def _dense1(deg2t, x_p, W1, n_pad):
    grid = n_pad // RB

    def body(deg_ref, x_ref, w_ref, u_ref, dv_ref):
        deg = deg_ref[:, 0:1] + deg_ref[:, 1:2] + 1.0
        dinv = 1.0 / jnp.sqrt(deg)
        g = jnp.dot(x_ref[...], w_ref[...], preferred_element_type=jnp.float32, precision=lax.Precision.HIGHEST)
        u_ref[...] = g * dinv
        dv_ref[...] = jnp.broadcast_to(dinv, (RB, H))

    return pl.pallas_call(
        body,
        grid=(grid,),
        in_specs=[
            pl.BlockSpec((RB, NC), lambda i: (i, 0)),
            pl.BlockSpec((RB, 2), lambda i: (i, 0)),
            pl.BlockSpec((2, H), lambda i: (0, 0)),
        ],
        out_specs=[
            pl.BlockSpec((RB, H), lambda i: (i, 0)),
            pl.BlockSpec((RB, H), lambda i: (i, 0)),
        ],
        out_shape=[
            jax.ShapeDtypeStruct((n_pad, H), jnp.float32),
            jax.ShapeDtypeStruct((n_pad, H), jnp.float32),
        ],
    )(deg2t, x_p, W1)


def _dense23(acc, u_prev, dinv8, b, W, n, n_pad):
    grid = n_pad // RB

    def body(acc_ref, u_ref, dv_ref, b_ref, w_ref, o_ref):
        i = pl.program_id(0)
        dv = dv_ref[...]
        t = (acc_ref[0] + acc_ref[1] + u_ref[...]) * dv + b_ref[...]
        h = _elu(t)
        un = jnp.dot(h, w_ref[...], preferred_element_type=jnp.float32, precision=lax.Precision.HIGHEST) * dv
        row = i * RB + lax.broadcasted_iota(jnp.int32, (RB, H), 0)
        o_ref[...] = jnp.where(row < n, un, 0.0)

    return pl.pallas_call(
        body,
        grid=(grid,),
        in_specs=[
            pl.BlockSpec((NC, RB, H), lambda i: (0, i, 0)),
            pl.BlockSpec((RB, H), lambda i: (i, 0)),
            pl.BlockSpec((RB, H), lambda i: (i, 0)),
            pl.BlockSpec((1, H), lambda i: (0, 0)),
            pl.BlockSpec((H, H), lambda i: (0, 0)),
        ],
        out_specs=pl.BlockSpec((RB, H), lambda i: (i, 0)),
        out_shape=jax.ShapeDtypeStruct((n_pad, H), jnp.float32),
    )(acc, u_prev, dinv8, b.reshape(1, H), W)


def _final(acc, u3, dinv8, b3, Wr1, br1, Wr2, br2, n, n_pad):
    grid = n_pad // RB

    def body(acc_ref, u_ref, dv_ref, b_ref, wr1_ref, br1_ref, wr2_ref, br2_ref,
             o_ref, accum):
        i = pl.program_id(0)
        t = (acc_ref[0] + acc_ref[1] + u_ref[...]) * dv_ref[...] + b_ref[...]
        h = _elu(t)
        row = i * RB + lax.broadcasted_iota(jnp.int32, (RB, H), 0)
        h = jnp.where(row < n, h, 0.0)
        part = jnp.sum(h, axis=0, keepdims=True)

        @pl.when(i == 0)
        def _():
            accum[...] = part

        @pl.when(i > 0)
        def _():
            accum[...] += part

        @pl.when(i == grid - 1)
        def _():
            m = accum[...] * (1.0 / n)
            v = jnp.dot(m, wr1_ref[...], preferred_element_type=jnp.float32, precision=lax.Precision.HIGHEST) + br1_ref[...]
            v = _elu(v)
            o_ref[...] = jnp.dot(v, wr2_ref[...], preferred_element_type=jnp.float32, precision=lax.Precision.HIGHEST) + br2_ref[...]

    return pl.pallas_call(
        body,
        grid=(grid,),
        in_specs=[
            pl.BlockSpec((NC, RB, H), lambda i: (0, i, 0)),
            pl.BlockSpec((RB, H), lambda i: (i, 0)),
            pl.BlockSpec((RB, H), lambda i: (i, 0)),
            pl.BlockSpec((1, H), lambda i: (0, 0)),
            pl.BlockSpec((H, H), lambda i: (0, 0)),
            pl.BlockSpec((1, H), lambda i: (0, 0)),
            pl.BlockSpec((H, 1), lambda i: (0, 0)),
            pl.BlockSpec((1, 1), lambda i: (0, 0)),
        ],
        out_specs=pl.BlockSpec((1, 1), lambda i: (0, 0)),
        out_shape=jax.ShapeDtypeStruct((1, 1), jnp.float32),
        scratch_shapes=[pltpu.VMEM((1, H), jnp.float32)],
    )(acc, u3, dinv8, b3.reshape(1, H), Wr1, br1.reshape(1, H), Wr2,
      br2.reshape(1, 1))


def kernel(x, edge_index, W1, b1, W2, b2, W3, b3, Wr1, br1, Wr2, br2):
    n = x.shape[0]
    e = edge_index.shape[1]
    n_pad = ((n + 1 + RB - 1) // RB) * RB
    chunk_edges = NW * GPC * GRP
    cpw = (e + chunk_edges - 1) // chunk_edges
    e_pad = cpw * chunk_edges

    sent = jnp.full((e_pad - e,), n, dtype=jnp.int32)
    src2 = jnp.concatenate([edge_index[0], sent]).reshape(e_pad // GRP, GRP)
    dst2 = jnp.concatenate([edge_index[1], sent]).reshape(e_pad // GRP, GRP)
    x_p = jnp.zeros((n_pad, x.shape[1]), jnp.float32).at[:n].set(x)
    zeros_n = jnp.zeros((n_pad,), jnp.float32)
    zeros_n8 = jnp.zeros((n_pad, H), jnp.float32)

    deg2 = _sc_degree(dst2, zeros_n, n_pad, cpw)
    u1, dinv8 = _dense1(deg2.T, x_p, W1, n_pad)

    acc = _sc_edge(u1, src2, dst2, zeros_n8, n_pad, cpw)
    u2 = _dense23(acc, u1, dinv8, b1, W2, n, n_pad)
    acc = _sc_edge(u2, src2, dst2, zeros_n8, n_pad, cpw)
    u3 = _dense23(acc, u2, dinv8, b2, W3, n, n_pad)
    acc = _sc_edge(u3, src2, dst2, zeros_n8, n_pad, cpw)

    return _final(acc, u3, dinv8, b3, Wr1, br1, Wr2, br2, n, n_pad)


# speedup vs baseline: 122.8429x; 1.4539x over previous
import jax
import jax.numpy as jnp
from jax import lax
from jax.experimental import pallas as pl
from jax.experimental.pallas import tpu as pltpu
from jax.experimental.pallas import tpu_sc as plsc

NC = 2
NS = 16
NW = NC * NS
GRP = 128
GPC = 4
NSL = 4
H = 8
RB = 2048


def _sc_degree(dst2, zeros_n, n_pad, cpw):
    rows_n = n_pad // NS
    t_iters = cpw // NSL

    def body(dst_hbm, zero_hbm, out_hbm, deg_sh, onesv, idxv, *sems):
        isem = sems[:NSL]
        ssem = sems[NSL:]
        c = lax.axis_index("c")
        s = lax.axis_index("s")
        w = c * NS + s
        sl = pl.ds(s * rows_n, rows_n)
        pltpu.sync_copy(zero_hbm.at[sl], deg_sh.at[sl])
        for i in range(GRP // 16):
            onesv[pl.ds(i * 16, 16)] = jnp.ones((16,), jnp.float32)
        plsc.subcore_barrier()

        def fire_idx(pp, k, ch):
            pltpu.async_copy(dst_hbm.at[pl.ds((w * cpw + ch) * GPC, GPC)],
                             idxv.at[pp, k], isem[k])

        def wait_idx(pp, k):
            pltpu.make_async_copy(dst_hbm.at[pl.ds(0, GPC)],
                                  idxv.at[pp, k], isem[k]).wait()

        def fire_scatters(pp, k):
            for g in range(GPC):
                pltpu.async_copy(onesv, deg_sh.at[idxv.at[pp, k, g]],
                                 ssem[k], add=True)

        def wait_scatters(pp, k):
            for g in range(GPC):
                pltpu.make_async_copy(onesv, deg_sh.at[idxv.at[pp, k, g]],
                                      ssem[k]).wait()

        for k in range(NSL):
            fire_idx(0, k, k)

        @pl.loop(0, t_iters)
        def _iter(t):
            p = lax.rem(t, 2)
            q = 1 - p
            for k in range(NSL):
                @pl.when(t > 0)
                def _():
                    wait_scatters(q, k)
                wait_idx(p, k)
                fire_scatters(p, k)

                @pl.when(t + 1 < t_iters)
                def _():
                    fire_idx(q, k, (t + 1) * NSL + k)

        p_last = lax.rem(t_iters - 1, 2)
        for k in range(NSL):
            wait_scatters(p_last, k)

        plsc.subcore_barrier()
        pltpu.sync_copy(deg_sh.at[sl], out_hbm.at[c, sl])

    f = pl.kernel(
        body,
        out_type=jax.ShapeDtypeStruct((NC, n_pad), jnp.float32),
        mesh=plsc.VectorSubcoreMesh(core_axis_name="c", subcore_axis_name="s"),
        scratch_types=[
            pltpu.VMEM_SHARED((n_pad,), jnp.float32),
            pltpu.VMEM((GRP,), jnp.float32),
            pltpu.VMEM((2, NSL, GPC, GRP), jnp.int32),
        ] + [pltpu.SemaphoreType.DMA] * (2 * NSL),
        compiler_params=pltpu.CompilerParams(use_tc_tiling_on_sc=False),
    )
    return f(dst2, zeros_n)


def _sc_edge(u, src2, dst2, zeros_n8, n_pad, cpw):
    rows_n = n_pad // NS
    t_iters = cpw // NSL

    def body(u_hbm, src_hbm, dst_hbm, zero_hbm, out_hbm,
             u_sh, acc_sh, srcv, dstv, rowsv, *sems):
        isem = sems[:NSL]
        gsem = sems[NSL:2 * NSL]
        ssem = sems[2 * NSL:]
        c = lax.axis_index("c")
        s = lax.axis_index("s")
        w = c * NS + s
        sl = pl.ds(s * rows_n, rows_n)
        pltpu.sync_copy(u_hbm.at[sl], u_sh.at[sl])
        pltpu.sync_copy(zero_hbm.at[sl], acc_sh.at[sl])
        plsc.subcore_barrier()

        def fire_idx(pp, k, ch):
            row0 = (w * cpw + ch) * GPC
            pltpu.async_copy(src_hbm.at[pl.ds(row0, GPC)], srcv.at[pp, k],
                             isem[k])
            pltpu.async_copy(dst_hbm.at[pl.ds(row0, GPC)], dstv.at[pp, k],
                             isem[k])

        def wait_idx(pp, k):
            pltpu.make_async_copy(src_hbm.at[pl.ds(0, GPC)], srcv.at[pp, k],
                                  isem[k]).wait()
            pltpu.make_async_copy(dst_hbm.at[pl.ds(0, GPC)], dstv.at[pp, k],
                                  isem[k]).wait()

        def fire_gathers(pp, k):
            for g in range(GPC):
                pltpu.async_copy(u_sh.at[srcv.at[pp, k, g]], rowsv.at[k, g],
                                 gsem[k])

        def wait_gathers(pp, k):
            for g in range(GPC):
                pltpu.make_async_copy(u_sh.at[srcv.at[pp, k, g]],
                                      rowsv.at[k, g], gsem[k]).wait()

        def fire_scatters(pp, k):
            for g in range(GPC):
                pltpu.async_copy(rowsv.at[k, g], acc_sh.at[dstv.at[pp, k, g]],
                                 ssem[k], add=True)

        def wait_scatters(pp, k):
            for g in range(GPC):
                pltpu.make_async_copy(rowsv.at[k, g],
                                      acc_sh.at[dstv.at[pp, k, g]],
                                      ssem[k]).wait()

        for k in range(NSL):
            fire_idx(0, k, k)

        @pl.loop(0, t_iters)
        def _iter(t):
            p = lax.rem(t, 2)
            q = 1 - p
            for k in range(NSL):
                @pl.when(t > 0)
                def _():
                    wait_scatters(q, k)
                wait_idx(p, k)
                fire_gathers(p, k)
            for k in range(NSL):
                wait_gathers(p, k)
                fire_scatters(p, k)

                @pl.when(t + 1 < t_iters)
                def _():
                    fire_idx(q, k, (t + 1) * NSL + k)

        p_last = lax.rem(t_iters - 1, 2)
        for k in range(NSL):
            wait_scatters(p_last, k)

        plsc.subcore_barrier()
        pltpu.sync_copy(acc_sh.at[sl], out_hbm.at[c, sl])

    f = pl.kernel(
        body,
        out_type=jax.ShapeDtypeStruct((NC, n_pad, H), jnp.float32),
        mesh=plsc.VectorSubcoreMesh(core_axis_name="c", subcore_axis_name="s"),
        scratch_types=[
            pltpu.VMEM_SHARED((n_pad, H), jnp.float32),
            pltpu.VMEM_SHARED((n_pad, H), jnp.float32),
            pltpu.VMEM((2, NSL, GPC, GRP), jnp.int32),
            pltpu.VMEM((2, NSL, GPC, GRP), jnp.int32),
            pltpu.VMEM((NSL, GPC, GRP, H), jnp.float32),
        ] + [pltpu.SemaphoreType.DMA] * (3 * NSL),
        compiler_params=pltpu.CompilerParams(use_tc_tiling_on_sc=False),
    )
    return f(u, src2, dst2, zeros_n8)


def _elu(t):
    return jnp.where(t > 0, t, jnp.exp(t) - 1.0)


def _dense1(deg2t, x_p, W1, n_pad):
    grid = n_pad // RB

    def body(deg_ref, x_ref, w_ref, u_ref, dv_ref):
        deg = deg_ref[:, 0:1] + deg_ref[:, 1:2] + 1.0
        dinv = 1.0 / jnp.sqrt(deg)
        g = jnp.dot(x_ref[...], w_ref[...], preferred_element_type=jnp.float32,
                    precision=lax.Precision.HIGHEST)
        u_ref[...] = g * dinv
        dv_ref[...] = jnp.broadcast_to(dinv, (RB, H))

    return pl.pallas_call(
        body,
        grid=(grid,),
        in_specs=[
            pl.BlockSpec((RB, NC), lambda i: (i, 0)),
            pl.BlockSpec((RB, 2), lambda i: (i, 0)),
            pl.BlockSpec((2, H), lambda i: (0, 0)),
        ],
        out_specs=[
            pl.BlockSpec((RB, H), lambda i: (i, 0)),
            pl.BlockSpec((RB, H), lambda i: (i, 0)),
        ],
        out_shape=[
            jax.ShapeDtypeStruct((n_pad, H), jnp.float32),
            jax.ShapeDtypeStruct((n_pad, H), jnp.float32),
        ],
    )(deg2t, x_p, W1)


def _dense23(acc, u_prev, dinv8, b, W, n, n_pad):
    grid = n_pad // RB

    def body(acc_ref, u_ref, dv_ref, b_ref, w_ref, o_ref):
        i = pl.program_id(0)
        dv = dv_ref[...]
        t = (acc_ref[0] + acc_ref[1] + u_ref[...]) * dv + b_ref[...]
        h = _elu(t)
        un = jnp.dot(h, w_ref[...], preferred_element_type=jnp.float32,
                     precision=lax.Precision.HIGHEST) * dv
        row = i * RB + lax.broadcasted_iota(jnp.int32, (RB, H), 0)
        o_ref[...] = jnp.where(row < n, un, 0.0)

    return pl.pallas_call(
        body,
        grid=(grid,),
        in_specs=[
            pl.BlockSpec((NC, RB, H), lambda i: (0, i, 0)),
            pl.BlockSpec((RB, H), lambda i: (i, 0)),
            pl.BlockSpec((RB, H), lambda i: (i, 0)),
            pl.BlockSpec((1, H), lambda i: (0, 0)),
            pl.BlockSpec((H, H), lambda i: (0, 0)),
        ],
        out_specs=pl.BlockSpec((RB, H), lambda i: (i, 0)),
        out_shape=jax.ShapeDtypeStruct((n_pad, H), jnp.float32),
    )(acc, u_prev, dinv8, b.reshape(1, H), W)


def _final(acc, u3, dinv8, b3, Wr1, br1, Wr2, br2, n, n_pad):
    grid = n_pad // RB

    def body(acc_ref, u_ref, dv_ref, b_ref, wr1_ref, br1_ref, wr2_ref, br2_ref,
             o_ref, accum):
        i = pl.program_id(0)
        t = (acc_ref[0] + acc_ref[1] + u_ref[...]) * dv_ref[...] + b_ref[...]
        h = _elu(t)
        row = i * RB + lax.broadcasted_iota(jnp.int32, (RB, H), 0)
        h = jnp.where(row < n, h, 0.0)
        part = jnp.sum(h, axis=0, keepdims=True)

        @pl.when(i == 0)
        def _():
            accum[...] = part

        @pl.when(i > 0)
        def _():
            accum[...] += part

        @pl.when(i == grid - 1)
        def _():
            m = accum[...] * (1.0 / n)
            v = jnp.dot(m, wr1_ref[...], preferred_element_type=jnp.float32,
                        precision=lax.Precision.HIGHEST) + br1_ref[...]
            v = _elu(v)
            o_ref[...] = jnp.dot(v, wr2_ref[...],
                                 preferred_element_type=jnp.float32,
                                 precision=lax.Precision.HIGHEST) + br2_ref[...]

    return pl.pallas_call(
        body,
        grid=(grid,),
        in_specs=[
            pl.BlockSpec((NC, RB, H), lambda i: (0, i, 0)),
            pl.BlockSpec((RB, H), lambda i: (i, 0)),
            pl.BlockSpec((RB, H), lambda i: (i, 0)),
            pl.BlockSpec((1, H), lambda i: (0, 0)),
            pl.BlockSpec((H, H), lambda i: (0, 0)),
            pl.BlockSpec((1, H), lambda i: (0, 0)),
            pl.BlockSpec((H, 1), lambda i: (0, 0)),
            pl.BlockSpec((1, 1), lambda i: (0, 0)),
        ],
        out_specs=pl.BlockSpec((1, 1), lambda i: (0, 0)),
        out_shape=jax.ShapeDtypeStruct((1, 1), jnp.float32),
        scratch_shapes=[pltpu.VMEM((1, H), jnp.float32)],
    )(acc, u3, dinv8, b3.reshape(1, H), Wr1, br1.reshape(1, H), Wr2,
      br2.reshape(1, 1))


def kernel(x, edge_index, W1, b1, W2, b2, W3, b3, Wr1, br1, Wr2, br2):
    n = x.shape[0]
    e = edge_index.shape[1]
    n_pad = ((n + 1 + RB - 1) // RB) * RB
    chunk_edges = NW * GPC * GRP
    cpw = (e + NSL * chunk_edges - 1) // (NSL * chunk_edges) * NSL
    e_pad = cpw * chunk_edges

    sent = jnp.full((e_pad - e,), n, dtype=jnp.int32)
    src2 = jnp.concatenate([edge_index[0], sent]).reshape(e_pad // GRP, GRP)
    dst2 = jnp.concatenate([edge_index[1], sent]).reshape(e_pad // GRP, GRP)
    x_p = jnp.zeros((n_pad, x.shape[1]), jnp.float32).at[:n].set(x)
    zeros_n = jnp.zeros((n_pad,), jnp.float32)
    zeros_n8 = jnp.zeros((n_pad, H), jnp.float32)

    deg2 = _sc_degree(dst2, zeros_n, n_pad, cpw)
    u1, dinv8 = _dense1(deg2.T, x_p, W1, n_pad)

    acc = _sc_edge(u1, src2, dst2, zeros_n8, n_pad, cpw)
    u2 = _dense23(acc, u1, dinv8, b1, W2, n, n_pad)
    acc = _sc_edge(u2, src2, dst2, zeros_n8, n_pad, cpw)
    u3 = _dense23(acc, u2, dinv8, b2, W3, n, n_pad)
    acc = _sc_edge(u3, src2, dst2, zeros_n8, n_pad, cpw)

    return _final(acc, u3, dinv8, b3, Wr1, br1, Wr2, br2, n, n_pad)


# speedup vs baseline: 219.9292x; 1.7903x over previous
import jax
import jax.numpy as jnp
from jax import lax
from jax.experimental import pallas as pl
from jax.experimental.pallas import tpu as pltpu
from jax.experimental.pallas import tpu_sc as plsc

NC = 2
NS = 16
NW = NC * NS
GRP = 128
GPC = 4
NSL = 4
H = 8
RB = 2048


def _sc_degree(dst2, ones_hbm_in, zeros_n8, n_pad, cpw):
    rows_n = n_pad // NS
    t_iters = cpw // NSL

    def body(dst_hbm, ones_hbm, zero_hbm, out_hbm, deg_sh, onesv, idxv, *sems):
        isem = sems[:NSL]
        ssem = sems[NSL:]
        c = lax.axis_index("c")
        s = lax.axis_index("s")
        w = c * NS + s
        sl = pl.ds(s * rows_n, rows_n)
        pltpu.sync_copy(zero_hbm.at[sl], deg_sh.at[sl])
        pltpu.sync_copy(ones_hbm, onesv)
        plsc.subcore_barrier()

        def fire_idx(pp, k, ch):
            pltpu.async_copy(dst_hbm.at[pl.ds((w * cpw + ch) * GPC, GPC)],
                             idxv.at[pp, k], isem[k])

        def wait_idx(pp, k):
            pltpu.make_async_copy(dst_hbm.at[pl.ds(0, GPC)],
                                  idxv.at[pp, k], isem[k]).wait()

        def fire_scatters(pp, k):
            for g in range(GPC):
                pltpu.async_copy(onesv, deg_sh.at[idxv.at[pp, k, g]],
                                 ssem[k], add=True)

        def wait_scatters(pp, k):
            for g in range(GPC):
                pltpu.make_async_copy(onesv, deg_sh.at[idxv.at[pp, k, g]],
                                      ssem[k]).wait()

        for k in range(NSL):
            fire_idx(0, k, k)

        @pl.loop(0, t_iters)
        def _iter(t):
            p = lax.rem(t, 2)
            q = 1 - p
            for k in range(NSL):
                @pl.when(t > 0)
                def _():
                    wait_scatters(q, k)
                wait_idx(p, k)
                fire_scatters(p, k)

                @pl.when(t + 1 < t_iters)
                def _():
                    fire_idx(q, k, (t + 1) * NSL + k)

        p_last = lax.rem(t_iters - 1, 2)
        for k in range(NSL):
            wait_scatters(p_last, k)

        plsc.subcore_barrier()
        pltpu.sync_copy(deg_sh.at[sl], out_hbm.at[c, sl])

    f = pl.kernel(
        body,
        out_type=jax.ShapeDtypeStruct((NC, n_pad, H), jnp.float32),
        mesh=plsc.VectorSubcoreMesh(core_axis_name="c", subcore_axis_name="s"),
        scratch_types=[
            pltpu.VMEM_SHARED((n_pad, H), jnp.float32),
            pltpu.VMEM((GRP, H), jnp.float32),
            pltpu.VMEM((2, NSL, GPC, GRP), jnp.int32),
        ] + [pltpu.SemaphoreType.DMA] * (2 * NSL),
        compiler_params=pltpu.CompilerParams(use_tc_tiling_on_sc=False),
    )
    return f(dst2, ones_hbm_in, zeros_n8)


def _sc_edge(u, src2, dst2, zeros_n8, n_pad, cpw):
    rows_n = n_pad // NS
    t_iters = cpw // NSL

    def body(u_hbm, src_hbm, dst_hbm, zero_hbm, out_hbm,
             u_sh, acc_sh, srcv, dstv, rowsv, *sems):
        isem = sems[:NSL]
        gsem = sems[NSL:2 * NSL]
        ssem = sems[2 * NSL:]
        c = lax.axis_index("c")
        s = lax.axis_index("s")
        w = c * NS + s
        sl = pl.ds(s * rows_n, rows_n)
        pltpu.sync_copy(u_hbm.at[sl], u_sh.at[sl])
        pltpu.sync_copy(zero_hbm.at[sl], acc_sh.at[sl])
        plsc.subcore_barrier()

        def fire_idx(pp, k, ch):
            row0 = (w * cpw + ch) * GPC
            pltpu.async_copy(src_hbm.at[pl.ds(row0, GPC)], srcv.at[pp, k],
                             isem[k])
            pltpu.async_copy(dst_hbm.at[pl.ds(row0, GPC)], dstv.at[pp, k],
                             isem[k])

        def wait_idx(pp, k):
            pltpu.make_async_copy(src_hbm.at[pl.ds(0, GPC)], srcv.at[pp, k],
                                  isem[k]).wait()
            pltpu.make_async_copy(dst_hbm.at[pl.ds(0, GPC)], dstv.at[pp, k],
                                  isem[k]).wait()

        def fire_gathers(pp, k):
            for g in range(GPC):
                pltpu.async_copy(u_sh.at[srcv.at[pp, k, g]], rowsv.at[k, g],
                                 gsem[k])

        def wait_gathers(pp, k):
            for g in range(GPC):
                pltpu.make_async_copy(u_sh.at[srcv.at[pp, k, g]],
                                      rowsv.at[k, g], gsem[k]).wait()

        def fire_scatters(pp, k):
            for g in range(GPC):
                pltpu.async_copy(rowsv.at[k, g], acc_sh.at[dstv.at[pp, k, g]],
                                 ssem[k], add=True)

        def wait_scatters(pp, k):
            for g in range(GPC):
                pltpu.make_async_copy(rowsv.at[k, g],
                                      acc_sh.at[dstv.at[pp, k, g]],
                                      ssem[k]).wait()

        for k in range(NSL):
            fire_idx(0, k, k)

        @pl.loop(0, t_iters)
        def _iter(t):
            p = lax.rem(t, 2)
            q = 1 - p
            for k in range(NSL):
                @pl.when(t > 0)
                def _():
                    wait_scatters(q, k)
                wait_idx(p, k)
                fire_gathers(p, k)
            for k in range(NSL):
                wait_gathers(p, k)
                fire_scatters(p, k)

                @pl.when(t + 1 < t_iters)
                def _():
                    fire_idx(q, k, (t + 1) * NSL + k)

        p_last = lax.rem(t_iters - 1, 2)
        for k in range(NSL):
            wait_scatters(p_last, k)

        plsc.subcore_barrier()
        pltpu.sync_copy(acc_sh.at[sl], out_hbm.at[c, sl])

    f = pl.kernel(
        body,
        out_type=jax.ShapeDtypeStruct((NC, n_pad, H), jnp.float32),
        mesh=plsc.VectorSubcoreMesh(core_axis_name="c", subcore_axis_name="s"),
        scratch_types=[
            pltpu.VMEM_SHARED((n_pad, H), jnp.float32),
            pltpu.VMEM_SHARED((n_pad, H), jnp.float32),
            pltpu.VMEM((2, NSL, GPC, GRP), jnp.int32),
            pltpu.VMEM((2, NSL, GPC, GRP), jnp.int32),
            pltpu.VMEM((NSL, GPC, GRP, H), jnp.float32),
        ] + [pltpu.SemaphoreType.DMA] * (3 * NSL),
        compiler_params=pltpu.CompilerParams(use_tc_tiling_on_sc=False),
    )
    return f(u, src2, dst2, zeros_n8)


RBF = 784


def _elu(t):
    return jnp.where(t > 0, t, jnp.exp(t) - 1.0)


def _dense1(deg_fat, x_fat, M1, npf):
    grid = npf // RBF

    def body(deg_ref, x_ref, m_ref, u_ref, dv_ref):
        deg = deg_ref[0] + deg_ref[1] + 1.0
        dinv = 1.0 / jnp.sqrt(deg)
        g = jnp.dot(x_ref[...], m_ref[...], preferred_element_type=jnp.float32,
                    precision=lax.Precision.HIGHEST)
        u_ref[...] = g * dinv
        dv_ref[...] = dinv

    return pl.pallas_call(
        body,
        grid=(grid,),
        in_specs=[
            pl.BlockSpec((NC, RBF, 128), lambda i: (0, i, 0)),
            pl.BlockSpec((RBF, 128), lambda i: (i, 0)),
            pl.BlockSpec((128, 128), lambda i: (0, 0)),
        ],
        out_specs=[
            pl.BlockSpec((RBF, 128), lambda i: (i, 0)),
            pl.BlockSpec((RBF, 128), lambda i: (i, 0)),
        ],
        out_shape=[
            jax.ShapeDtypeStruct((npf, 128), jnp.float32),
            jax.ShapeDtypeStruct((npf, 128), jnp.float32),
        ],
    )(deg_fat, x_fat, M1)


def _dense23(acc_fat, u_prev, dv_fat, b, M, n, npf):
    grid = npf // RBF
    b128 = jnp.tile(b, 16).reshape(1, 128)

    def body(acc_ref, u_ref, dv_ref, b_ref, m_ref, o_ref):
        i = pl.program_id(0)
        dv = dv_ref[...]
        t = (acc_ref[0] + acc_ref[1] + u_ref[...]) * dv + b_ref[...]
        h = _elu(t)
        un = jnp.dot(h, m_ref[...], preferred_element_type=jnp.float32,
                     precision=lax.Precision.HIGHEST) * dv
        flat = ((i * RBF + lax.broadcasted_iota(jnp.int32, (RBF, 128), 0)) * 128
                + lax.broadcasted_iota(jnp.int32, (RBF, 128), 1))
        o_ref[...] = jnp.where(flat < H * n, un, 0.0)

    return pl.pallas_call(
        body,
        grid=(grid,),
        in_specs=[
            pl.BlockSpec((NC, RBF, 128), lambda i: (0, i, 0)),
            pl.BlockSpec((RBF, 128), lambda i: (i, 0)),
            pl.BlockSpec((RBF, 128), lambda i: (i, 0)),
            pl.BlockSpec((1, 128), lambda i: (0, 0)),
            pl.BlockSpec((128, 128), lambda i: (0, 0)),
        ],
        out_specs=pl.BlockSpec((RBF, 128), lambda i: (i, 0)),
        out_shape=jax.ShapeDtypeStruct((npf, 128), jnp.float32),
    )(acc_fat, u_prev, dv_fat, b128, M)


def _final(acc_fat, u3, dv_fat, R, b3, Wr1, br1, Wr2, br2, n, npf):
    grid = npf // RBF
    b128 = jnp.tile(b3, 16).reshape(1, 128)

    def body(acc_ref, u_ref, dv_ref, b_ref, r_ref, wr1_ref, br1_ref, wr2_ref,
             br2_ref, o_ref, accum):
        i = pl.program_id(0)
        t = (acc_ref[0] + acc_ref[1] + u_ref[...]) * dv_ref[...] + b_ref[...]
        h = _elu(t)
        flat = ((i * RBF + lax.broadcasted_iota(jnp.int32, (RBF, 128), 0)) * 128
                + lax.broadcasted_iota(jnp.int32, (RBF, 128), 1))
        h = jnp.where(flat < H * n, h, 0.0)
        part = jnp.sum(h, axis=0, keepdims=True)

        @pl.when(i == 0)
        def _():
            accum[...] = part

        @pl.when(i > 0)
        def _():
            accum[...] += part

        @pl.when(i == grid - 1)
        def _():
            m = jnp.dot(accum[...], r_ref[...],
                        preferred_element_type=jnp.float32,
                        precision=lax.Precision.HIGHEST) * (1.0 / n)
            v = jnp.dot(m, wr1_ref[...], preferred_element_type=jnp.float32,
                        precision=lax.Precision.HIGHEST) + br1_ref[...]
            v = _elu(v)
            o_ref[...] = jnp.dot(v, wr2_ref[...],
                                 preferred_element_type=jnp.float32,
                                 precision=lax.Precision.HIGHEST) + br2_ref[...]

    return pl.pallas_call(
        body,
        grid=(grid,),
        in_specs=[
            pl.BlockSpec((NC, RBF, 128), lambda i: (0, i, 0)),
            pl.BlockSpec((RBF, 128), lambda i: (i, 0)),
            pl.BlockSpec((RBF, 128), lambda i: (i, 0)),
            pl.BlockSpec((1, 128), lambda i: (0, 0)),
            pl.BlockSpec((128, H), lambda i: (0, 0)),
            pl.BlockSpec((H, H), lambda i: (0, 0)),
            pl.BlockSpec((1, H), lambda i: (0, 0)),
            pl.BlockSpec((H, 1), lambda i: (0, 0)),
            pl.BlockSpec((1, 1), lambda i: (0, 0)),
        ],
        out_specs=pl.BlockSpec((1, 1), lambda i: (0, 0)),
        out_shape=jax.ShapeDtypeStruct((1, 1), jnp.float32),
        scratch_shapes=[pltpu.VMEM((1, 128), jnp.float32)],
    )(acc_fat, u3, dv_fat, b128, R, Wr1, br1.reshape(1, H), Wr2,
      br2.reshape(1, 1))


def kernel(x, edge_index, W1, b1, W2, b2, W3, b3, Wr1, br1, Wr2, br2):
    n = x.shape[0]
    e = edge_index.shape[1]
    n_pad = ((n + 1 + 16 * RBF - 1) // (16 * RBF)) * (16 * RBF)
    npf = n_pad // 16
    chunk_edges = NW * GPC * GRP
    cpw = (e + NSL * chunk_edges - 1) // (NSL * chunk_edges) * NSL
    e_pad = cpw * chunk_edges

    sent_src = jnp.full((e_pad - e,), n, dtype=jnp.int32)
    sent_dst = n + jnp.arange(e_pad - e, dtype=jnp.int32) % (n_pad - n)
    src2 = jnp.concatenate([edge_index[0], sent_src]).reshape(e_pad // GRP, GRP)
    dst2 = jnp.concatenate([edge_index[1], sent_dst]).reshape(e_pad // GRP, GRP)
    x_fat = (jnp.zeros((n_pad, H), jnp.float32).at[:n, :2].set(x)
             .reshape(npf, 128))
    zeros_n8 = jnp.zeros((n_pad, H), jnp.float32)
    ones128 = jnp.ones((GRP, H), jnp.float32)
    eye16 = jnp.eye(16, dtype=jnp.float32)
    W1p = jnp.zeros((H, H), jnp.float32).at[:2].set(W1)
    M1 = jnp.kron(eye16, W1p)
    M2 = jnp.kron(eye16, W2)
    M3 = jnp.kron(eye16, W3)
    R = jnp.tile(jnp.eye(H, dtype=jnp.float32), (16, 1))

    deg2 = _sc_degree(dst2, ones128, zeros_n8, n_pad, cpw)
    u1, dv_fat = _dense1(deg2.reshape(NC, npf, 128), x_fat, M1, npf)

    acc = _sc_edge(u1.reshape(n_pad, H), src2, dst2, zeros_n8, n_pad, cpw)
    u2 = _dense23(acc.reshape(NC, npf, 128), u1, dv_fat, b1, M2, n, npf)
    acc = _sc_edge(u2.reshape(n_pad, H), src2, dst2, zeros_n8, n_pad, cpw)
    u3 = _dense23(acc.reshape(NC, npf, 128), u2, dv_fat, b2, M3, n, npf)
    acc = _sc_edge(u3.reshape(n_pad, H), src2, dst2, zeros_n8, n_pad, cpw)

    return _final(acc.reshape(NC, npf, 128), u3, dv_fat, R, b3, Wr1, br1,
                  Wr2, br2, n, npf)


# speedup vs baseline: 220.2200x; 1.0013x over previous
import jax
import jax.numpy as jnp
from jax import lax
from jax.experimental import pallas as pl
from jax.experimental.pallas import tpu as pltpu
from jax.experimental.pallas import tpu_sc as plsc

NC = 2
NS = 16
NW = NC * NS
GRP = 128
GPC = 4
GPD = 8
NSL = 4
H = 8
RB = 2048


def _sc_degree(dst2, ones_hbm_in, zeros_n8, n_pad, cpw):
    rows_n = n_pad // NS
    t_iters = cpw // NSL

    def body(dst_hbm, ones_hbm, zero_hbm, out_hbm, deg_sh, onesv, idxv, *sems):
        isem = sems[:NSL]
        ssem = sems[NSL:]
        c = lax.axis_index("c")
        s = lax.axis_index("s")
        w = c * NS + s
        sl = pl.ds(s * rows_n, rows_n)
        pltpu.sync_copy(zero_hbm.at[sl], deg_sh.at[sl])
        pltpu.sync_copy(ones_hbm, onesv)
        plsc.subcore_barrier()

        def fire_idx(pp, k, ch):
            pltpu.async_copy(dst_hbm.at[pl.ds((w * cpw + ch) * GPD, GPD)],
                             idxv.at[pp, k], isem[k])

        def wait_idx(pp, k):
            pltpu.make_async_copy(dst_hbm.at[pl.ds(0, GPD)],
                                  idxv.at[pp, k], isem[k]).wait()

        def fire_scatters(pp, k):
            for g in range(GPD):
                pltpu.async_copy(onesv, deg_sh.at[idxv.at[pp, k, g]],
                                 ssem[k], add=True)

        def wait_scatters(pp, k):
            for g in range(GPD):
                pltpu.make_async_copy(onesv, deg_sh.at[idxv.at[pp, k, g]],
                                      ssem[k]).wait()

        for k in range(NSL):
            fire_idx(0, k, k)

        @pl.loop(0, t_iters)
        def _iter(t):
            p = lax.rem(t, 2)
            q = 1 - p
            for k in range(NSL):
                @pl.when(t > 0)
                def _():
                    wait_scatters(q, k)
                wait_idx(p, k)
                fire_scatters(p, k)

                @pl.when(t + 1 < t_iters)
                def _():
                    fire_idx(q, k, (t + 1) * NSL + k)

        p_last = lax.rem(t_iters - 1, 2)
        for k in range(NSL):
            wait_scatters(p_last, k)

        plsc.subcore_barrier()
        pltpu.sync_copy(deg_sh.at[sl], out_hbm.at[c, sl])

    f = pl.kernel(
        body,
        out_type=jax.ShapeDtypeStruct((NC, n_pad, H), jnp.float32),
        mesh=plsc.VectorSubcoreMesh(core_axis_name="c", subcore_axis_name="s"),
        scratch_types=[
            pltpu.VMEM_SHARED((n_pad, H), jnp.float32),
            pltpu.VMEM((GRP, H), jnp.float32),
            pltpu.VMEM((2, NSL, GPD, GRP), jnp.int32),
        ] + [pltpu.SemaphoreType.DMA] * (2 * NSL),
        compiler_params=pltpu.CompilerParams(use_tc_tiling_on_sc=False),
    )
    return f(dst2, ones_hbm_in, zeros_n8)


def _sc_edge(u, src2, dst2, zeros_n8, n_pad, cpw):
    rows_n = n_pad // NS
    t_iters = cpw // NSL

    def body(u_hbm, src_hbm, dst_hbm, zero_hbm, out_hbm,
             u_sh, acc_sh, srcv, dstv, rowsv, *sems):
        isem = sems[:NSL]
        gsem = sems[NSL:2 * NSL]
        ssem = sems[2 * NSL:]
        c = lax.axis_index("c")
        s = lax.axis_index("s")
        w = c * NS + s
        sl = pl.ds(s * rows_n, rows_n)
        pltpu.sync_copy(u_hbm.at[sl], u_sh.at[sl])
        pltpu.sync_copy(zero_hbm.at[sl], acc_sh.at[sl])
        plsc.subcore_barrier()

        def fire_idx(pp, k, ch):
            row0 = (w * cpw + ch) * GPC
            pltpu.async_copy(src_hbm.at[pl.ds(row0, GPC)], srcv.at[pp, k],
                             isem[k])
            pltpu.async_copy(dst_hbm.at[pl.ds(row0, GPC)], dstv.at[pp, k],
                             isem[k])

        def wait_idx(pp, k):
            pltpu.make_async_copy(src_hbm.at[pl.ds(0, GPC)], srcv.at[pp, k],
                                  isem[k]).wait()
            pltpu.make_async_copy(dst_hbm.at[pl.ds(0, GPC)], dstv.at[pp, k],
                                  isem[k]).wait()

        def fire_gathers(pp, k):
            for g in range(GPC):
                pltpu.async_copy(u_sh.at[srcv.at[pp, k, g]], rowsv.at[k, g],
                                 gsem[k])

        def wait_gathers(pp, k):
            for g in range(GPC):
                pltpu.make_async_copy(u_sh.at[srcv.at[pp, k, g]],
                                      rowsv.at[k, g], gsem[k]).wait()

        def fire_scatters(pp, k):
            for g in range(GPC):
                pltpu.async_copy(rowsv.at[k, g], acc_sh.at[dstv.at[pp, k, g]],
                                 ssem[k], add=True)

        def wait_scatters(pp, k):
            for g in range(GPC):
                pltpu.make_async_copy(rowsv.at[k, g],
                                      acc_sh.at[dstv.at[pp, k, g]],
                                      ssem[k]).wait()

        for k in range(NSL):
            fire_idx(0, k, k)

        @pl.loop(0, t_iters)
        def _iter(t):
            p = lax.rem(t, 2)
            q = 1 - p
            for k in range(NSL):
                @pl.when(t > 0)
                def _():
                    wait_scatters(q, k)
                wait_idx(p, k)
                fire_gathers(p, k)
            for k in range(NSL):
                wait_gathers(p, k)
                fire_scatters(p, k)

                @pl.when(t + 1 < t_iters)
                def _():
                    fire_idx(q, k, (t + 1) * NSL + k)

        p_last = lax.rem(t_iters - 1, 2)
        for k in range(NSL):
            wait_scatters(p_last, k)

        plsc.subcore_barrier()
        pltpu.sync_copy(acc_sh.at[sl], out_hbm.at[c, sl])

    f = pl.kernel(
        body,
        out_type=jax.ShapeDtypeStruct((NC, n_pad, H), jnp.float32),
        mesh=plsc.VectorSubcoreMesh(core_axis_name="c", subcore_axis_name="s"),
        scratch_types=[
            pltpu.VMEM_SHARED((n_pad, H), jnp.float32),
            pltpu.VMEM_SHARED((n_pad, H), jnp.float32),
            pltpu.VMEM((2, NSL, GPC, GRP), jnp.int32),
            pltpu.VMEM((2, NSL, GPC, GRP), jnp.int32),
            pltpu.VMEM((NSL, GPC, GRP, H), jnp.float32),
        ] + [pltpu.SemaphoreType.DMA] * (3 * NSL),
        compiler_params=pltpu.CompilerParams(use_tc_tiling_on_sc=False),
    )
    return f(u, src2, dst2, zeros_n8)


RBF = 784


def _elu(t):
    return jnp.where(t > 0, t, jnp.exp(t) - 1.0)


def _dense1(deg_fat, x_fat, M1, npf):
    grid = npf // RBF

    def body(deg_ref, x_ref, m_ref, u_ref, dv_ref):
        deg = deg_ref[0] + deg_ref[1] + 1.0
        dinv = 1.0 / jnp.sqrt(deg)
        g = jnp.dot(x_ref[...], m_ref[...], preferred_element_type=jnp.float32,
                    precision=lax.Precision.HIGHEST)
        u_ref[...] = g * dinv
        dv_ref[...] = dinv

    return pl.pallas_call(
        body,
        grid=(grid,),
        in_specs=[
            pl.BlockSpec((NC, RBF, 128), lambda i: (0, i, 0)),
            pl.BlockSpec((RBF, 128), lambda i: (i, 0)),
            pl.BlockSpec((128, 128), lambda i: (0, 0)),
        ],
        out_specs=[
            pl.BlockSpec((RBF, 128), lambda i: (i, 0)),
            pl.BlockSpec((RBF, 128), lambda i: (i, 0)),
        ],
        out_shape=[
            jax.ShapeDtypeStruct((npf, 128), jnp.float32),
            jax.ShapeDtypeStruct((npf, 128), jnp.float32),
        ],
    )(deg_fat, x_fat, M1)


def _dense23(acc_fat, u_prev, dv_fat, b, M, n, npf):
    grid = npf // RBF
    b128 = jnp.tile(b, 16).reshape(1, 128)

    def body(acc_ref, u_ref, dv_ref, b_ref, m_ref, o_ref):
        i = pl.program_id(0)
        dv = dv_ref[...]
        t = (acc_ref[0] + acc_ref[1] + u_ref[...]) * dv + b_ref[...]
        h = _elu(t)
        un = jnp.dot(h, m_ref[...], preferred_element_type=jnp.float32,
                     precision=lax.Precision.HIGHEST) * dv
        flat = ((i * RBF + lax.broadcasted_iota(jnp.int32, (RBF, 128), 0)) * 128
                + lax.broadcasted_iota(jnp.int32, (RBF, 128), 1))
        o_ref[...] = jnp.where(flat < H * n, un, 0.0)

    return pl.pallas_call(
        body,
        grid=(grid,),
        in_specs=[
            pl.BlockSpec((NC, RBF, 128), lambda i: (0, i, 0)),
            pl.BlockSpec((RBF, 128), lambda i: (i, 0)),
            pl.BlockSpec((RBF, 128), lambda i: (i, 0)),
            pl.BlockSpec((1, 128), lambda i: (0, 0)),
            pl.BlockSpec((128, 128), lambda i: (0, 0)),
        ],
        out_specs=pl.BlockSpec((RBF, 128), lambda i: (i, 0)),
        out_shape=jax.ShapeDtypeStruct((npf, 128), jnp.float32),
    )(acc_fat, u_prev, dv_fat, b128, M)


def _final(acc_fat, u3, dv_fat, R, b3, Wr1, br1, Wr2, br2, n, npf):
    grid = npf // RBF
    b128 = jnp.tile(b3, 16).reshape(1, 128)

    def body(acc_ref, u_ref, dv_ref, b_ref, r_ref, wr1_ref, br1_ref, wr2_ref,
             br2_ref, o_ref, accum):
        i = pl.program_id(0)
        t = (acc_ref[0] + acc_ref[1] + u_ref[...]) * dv_ref[...] + b_ref[...]
        h = _elu(t)
        flat = ((i * RBF + lax.broadcasted_iota(jnp.int32, (RBF, 128), 0)) * 128
                + lax.broadcasted_iota(jnp.int32, (RBF, 128), 1))
        h = jnp.where(flat < H * n, h, 0.0)
        part = jnp.sum(h, axis=0, keepdims=True)

        @pl.when(i == 0)
        def _():
            accum[...] = part

        @pl.when(i > 0)
        def _():
            accum[...] += part

        @pl.when(i == grid - 1)
        def _():
            m = jnp.dot(accum[...], r_ref[...],
                        preferred_element_type=jnp.float32,
                        precision=lax.Precision.HIGHEST) * (1.0 / n)
            v = jnp.dot(m, wr1_ref[...], preferred_element_type=jnp.float32,
                        precision=lax.Precision.HIGHEST) + br1_ref[...]
            v = _elu(v)
            o_ref[...] = jnp.dot(v, wr2_ref[...],
                                 preferred_element_type=jnp.float32,
                                 precision=lax.Precision.HIGHEST) + br2_ref[...]

    return pl.pallas_call(
        body,
        grid=(grid,),
        in_specs=[
            pl.BlockSpec((NC, RBF, 128), lambda i: (0, i, 0)),
            pl.BlockSpec((RBF, 128), lambda i: (i, 0)),
            pl.BlockSpec((RBF, 128), lambda i: (i, 0)),
            pl.BlockSpec((1, 128), lambda i: (0, 0)),
            pl.BlockSpec((128, H), lambda i: (0, 0)),
            pl.BlockSpec((H, H), lambda i: (0, 0)),
            pl.BlockSpec((1, H), lambda i: (0, 0)),
            pl.BlockSpec((H, 1), lambda i: (0, 0)),
            pl.BlockSpec((1, 1), lambda i: (0, 0)),
        ],
        out_specs=pl.BlockSpec((1, 1), lambda i: (0, 0)),
        out_shape=jax.ShapeDtypeStruct((1, 1), jnp.float32),
        scratch_shapes=[pltpu.VMEM((1, 128), jnp.float32)],
    )(acc_fat, u3, dv_fat, b128, R, Wr1, br1.reshape(1, H), Wr2,
      br2.reshape(1, 1))


def kernel(x, edge_index, W1, b1, W2, b2, W3, b3, Wr1, br1, Wr2, br2):
    n = x.shape[0]
    e = edge_index.shape[1]
    n_pad = ((n + 1 + 16 * RBF - 1) // (16 * RBF)) * (16 * RBF)
    npf = n_pad // 16
    unit = NSL * NW * GPD * GRP
    e_pad = (e + unit - 1) // unit * unit
    cpw = e_pad // (NW * GPC * GRP)
    cpw_d = e_pad // (NW * GPD * GRP)

    sent_src = jnp.full((e_pad - e,), n, dtype=jnp.int32)
    sent_dst = n + jnp.arange(e_pad - e, dtype=jnp.int32) % (n_pad - n)
    src2 = jnp.concatenate([edge_index[0], sent_src]).reshape(e_pad // GRP, GRP)
    dst2 = jnp.concatenate([edge_index[1], sent_dst]).reshape(e_pad // GRP, GRP)
    x_fat = (jnp.zeros((n_pad, H), jnp.float32).at[:n, :2].set(x)
             .reshape(npf, 128))
    zeros_n8 = jnp.zeros((n_pad, H), jnp.float32)
    ones128 = jnp.ones((GRP, H), jnp.float32)
    eye16 = jnp.eye(16, dtype=jnp.float32)
    W1p = jnp.zeros((H, H), jnp.float32).at[:2].set(W1)
    M1 = jnp.kron(eye16, W1p)
    M2 = jnp.kron(eye16, W2)
    M3 = jnp.kron(eye16, W3)
    R = jnp.tile(jnp.eye(H, dtype=jnp.float32), (16, 1))

    deg2 = _sc_degree(dst2, ones128, zeros_n8, n_pad, cpw_d)
    u1, dv_fat = _dense1(deg2.reshape(NC, npf, 128), x_fat, M1, npf)

    acc = _sc_edge(u1.reshape(n_pad, H), src2, dst2, zeros_n8, n_pad, cpw)
    u2 = _dense23(acc.reshape(NC, npf, 128), u1, dv_fat, b1, M2, n, npf)
    acc = _sc_edge(u2.reshape(n_pad, H), src2, dst2, zeros_n8, n_pad, cpw)
    u3 = _dense23(acc.reshape(NC, npf, 128), u2, dv_fat, b2, M3, n, npf)
    acc = _sc_edge(u3.reshape(n_pad, H), src2, dst2, zeros_n8, n_pad, cpw)

    return _final(acc.reshape(NC, npf, 128), u3, dv_fat, R, b3, Wr1, br1,
                  Wr2, br2, n, npf)
